# Initial kernel scaffold; baseline (speedup 1.0000x reference)
import jax
import jax.numpy as jnp
from jax.experimental import pallas as pl


def kernel(x, edge_index, edge_weight, W1, b1, W2, b2):
    raise NotImplementedError("write your pallas kernel here")


# speedup vs baseline: 8.2481x; 8.2481x over previous
import functools

import jax
import jax.numpy as jnp
from jax import lax
from jax.experimental import pallas as pl
from jax.experimental.pallas import tpu as pltpu
from jax.experimental.pallas import tpu_sc as plsc

NC = 2
NS = 16
LANES = 16
K = 128
BM = 1024


def _cdiv(a, b):
    return (a + b - 1) // b


def _make_deg_kernel(Np, Ep, F):
    cpt = Ep // (NC * NS * K)
    rpt = Np // NS // K
    nf = F // LANES
    mesh = plsc.VectorSubcoreMesh(
        core_axis_name="c", subcore_axis_name="s", num_cores=NC, num_subcores=NS)

    @functools.partial(
        pl.kernel,
        out_type=jax.ShapeDtypeStruct((NC, Np, F), jnp.float32),
        mesh=mesh,
        scratch_types=[
            pltpu.VMEM((K,), jnp.int32),
            pltpu.VMEM((K,), jnp.float32),
            pltpu.VMEM((K, F), jnp.float32),
            pltpu.VMEM_SHARED((Np, F), jnp.float32),
        ],
    )
    def deg_kernel(dst_hbm, ew_hbm, out_hbm, d_idx, w_v, rows_v, deg_sh):
        cid = lax.axis_index("c")
        sid = lax.axis_index("s")
        wid = cid * NS + sid

        zero = jnp.zeros((LANES,), jnp.float32)

        def zrow(k, _):
            for f in range(nf):
                rows_v[k, pl.ds(f * LANES, LANES)] = zero
            return _

        lax.fori_loop(0, K, zrow, None)
        for t in range(rpt):
            pltpu.sync_copy(rows_v, deg_sh.at[pl.ds(sid * (Np // NS) + t * K, K)])
        plsc.subcore_barrier()

        def chunk(j, _):
            off = (wid * cpt + j) * K
            pltpu.sync_copy(dst_hbm.at[pl.ds(off, K)], d_idx)
            pltpu.sync_copy(ew_hbm.at[pl.ds(off, K)], w_v)

            def fill(g, _):
                w16 = w_v[pl.ds(g * LANES, LANES)]
                for l in range(LANES):
                    wv = jnp.full((LANES,), w16[l], jnp.float32)
                    for f in range(nf):
                        rows_v[g * LANES + l, pl.ds(f * LANES, LANES)] = wv
                return _

            lax.fori_loop(0, K // LANES, fill, None)
            pltpu.sync_copy(rows_v, deg_sh.at[d_idx], add=True)
            return _

        lax.fori_loop(0, cpt, chunk, None)
        plsc.subcore_barrier()

        for t in range(rpt):
            r0 = sid * (Np // NS) + t * K
            pltpu.sync_copy(deg_sh.at[pl.ds(r0, K)], out_hbm.at[cid, pl.ds(r0, K)])

    return deg_kernel


def _make_agg_kernel(Np, Ep, F):
    cpt = Ep // (NC * NS * K)
    rpt = Np // NS // K
    nf = F // LANES
    mesh = plsc.VectorSubcoreMesh(
        core_axis_name="c", subcore_axis_name="s", num_cores=NC, num_subcores=NS)

    @functools.partial(
        pl.kernel,
        out_type=jax.ShapeDtypeStruct((NC, Np, F), jnp.float32),
        mesh=mesh,
        scratch_types=[
            pltpu.VMEM((K,), jnp.int32),
            pltpu.VMEM((K,), jnp.int32),
            pltpu.VMEM((K,), jnp.float32),
            pltpu.VMEM((K, F), jnp.float32),
            pltpu.VMEM_SHARED((Np, F), jnp.float32),
            pltpu.SemaphoreType.DMA,
        ],
    )
    def agg_kernel(y_hbm, src_hbm, dst_hbm, ew_hbm, out_hbm,
                   s_idx, d_idx, w_v, rows_v, acc_sh, sem):
        cid = lax.axis_index("c")
        sid = lax.axis_index("s")
        wid = cid * NS + sid

        zero = jnp.zeros((LANES,), jnp.float32)

        def zrow(k, _):
            for f in range(nf):
                rows_v[k, pl.ds(f * LANES, LANES)] = zero
            return _

        lax.fori_loop(0, K, zrow, None)
        for t in range(rpt):
            pltpu.sync_copy(rows_v, acc_sh.at[pl.ds(sid * (Np // NS) + t * K, K)])
        plsc.subcore_barrier()

        def chunk(j, _):
            off = (wid * cpt + j) * K
            pltpu.sync_copy(src_hbm.at[pl.ds(off, K)], s_idx)
            pltpu.sync_copy(dst_hbm.at[pl.ds(off, K)], d_idx)
            pltpu.sync_copy(ew_hbm.at[pl.ds(off, K)], w_v)
            pltpu.async_copy(y_hbm.at[s_idx], rows_v, sem).wait()

            def scale(g, _):
                w16 = w_v[pl.ds(g * LANES, LANES)]
                for l in range(LANES):
                    k = g * LANES + l
                    w = jnp.full((LANES,), w16[l], jnp.float32)
                    for f in range(nf):
                        sl = pl.ds(f * LANES, LANES)
                        rows_v[k, sl] = rows_v[k, sl] * w
                return _

            lax.fori_loop(0, K // LANES, scale, None)
            pltpu.sync_copy(rows_v, acc_sh.at[d_idx], add=True)
            return _

        lax.fori_loop(0, cpt, chunk, None)
        plsc.subcore_barrier()

        for t in range(rpt):
            r0 = sid * (Np // NS) + t * K
            pltpu.sync_copy(acc_sh.at[pl.ds(r0, K)], out_hbm.at[cid, pl.ds(r0, K)])

    return agg_kernel


def _dinv_from_deg(dp):
    deg = dp[0, :, 0] + dp[1, :, 0] + 1.0
    return jnp.where(deg > 0, lax.rsqrt(jnp.maximum(deg, 1e-12)), 0.0)


def _tc_first(x_ref, w_ref, dp_ref, y_ref):
    dinv = _dinv_from_deg(dp_ref[...])
    xw = jnp.dot(x_ref[...], w_ref[...], preferred_element_type=jnp.float32)
    y_ref[...] = dinv[:, None] * xw


def _tc_mid(a0_ref, a1_ref, y1_ref, dp_ref, b_ref, w_ref, y2_ref):
    dinv = _dinv_from_deg(dp_ref[...])
    pre = dinv[:, None] * (a0_ref[...] + a1_ref[...] + y1_ref[...]) + b_ref[...]
    h = jnp.where(pre >= 0, pre, 0.01 * pre)
    hw = jnp.dot(h, w_ref[...], preferred_element_type=jnp.float32)
    y2_ref[...] = dinv[:, None] * hw


def _tc_last(a0_ref, a1_ref, y2_ref, dp_ref, b_ref, out_ref):
    dinv = _dinv_from_deg(dp_ref[...])
    out_ref[...] = dinv[:, None] * (a0_ref[...] + a1_ref[...] + y2_ref[...]) + b_ref[...]


def _row_spec(F):
    return pl.BlockSpec((BM, F), lambda i: (i, 0))


def _tc_call(body, n_in, Np, F, extra_specs):
    return pl.pallas_call(
        body,
        grid=(Np // BM,),
        in_specs=extra_specs,
        out_specs=_row_spec(F),
        out_shape=jax.ShapeDtypeStruct((Np, F), jnp.float32),
    )


def kernel(x, edge_index, edge_weight, W1, b1, W2, b2):
    N, F = x.shape
    E = edge_index.shape[1]
    Np = _cdiv(N, NS * K) * NS * K
    Ep = _cdiv(E, NC * NS * K) * NC * NS * K

    src = edge_index[0]
    dst = edge_index[1]
    if Ep != E:
        pad = Ep - E
        src = jnp.pad(src, (0, pad))
        dst = jnp.pad(dst, (0, pad))
        edge_weight = jnp.pad(edge_weight, (0, pad))
    xp = jnp.pad(x, ((0, Np - N), (0, 0))) if Np != N else x
    b1r = b1.reshape(1, F)
    b2r = b2.reshape(1, F)

    deg_k = _make_deg_kernel(Np, Ep, F)
    agg_k = _make_agg_kernel(Np, Ep, F)

    degp = deg_k(dst, edge_weight)

    dp_spec = pl.BlockSpec((NC, BM, F), lambda i: (0, i, 0))
    w_spec = pl.BlockSpec((F, F), lambda i: (0, 0))
    b_spec = pl.BlockSpec((1, F), lambda i: (0, 0))
    row = _row_spec(F)

    y1 = _tc_call(_tc_first, 3, Np, F, [row, w_spec, dp_spec])(xp, W1, degp)

    acc1 = agg_k(y1, src, dst, edge_weight)

    y2 = _tc_call(_tc_mid, 6, Np, F, [row, row, row, dp_spec, b_spec, w_spec])(
        acc1[0], acc1[1], y1, degp, b1r, W2)

    acc2 = agg_k(y2, src, dst, edge_weight)

    out = _tc_call(_tc_last, 5, Np, F, [row, row, row, dp_spec, b_spec])(
        acc2[0], acc2[1], y2, degp, b2r)

    return out[:N]
